# Initial kernel scaffold; baseline (speedup 1.0000x reference)
import jax
import jax.numpy as jnp
from jax.experimental import pallas as pl


def kernel(inputs, emb_table, W, b):
    raise NotImplementedError("write your pallas kernel here")


# speedup vs baseline: 1.5175x; 1.5175x over previous
import functools

import jax
import jax.numpy as jnp
from jax import lax
from jax.experimental import pallas as pl
from jax.experimental.pallas import tpu as pltpu
from jax.experimental.pallas import tpu_sc as plsc

VOCAB = 100000
DIM = 32
CTX = 20
BATCH = 4096
CTXDIM = CTX * DIM

CHUNK = 128
NCHUNK = BATCH // CHUNK

VT = 2048
VTC = 12800


def _sc_gather_sum(idx_t, emb_table):
    mesh = plsc.VectorSubcoreMesh(core_axis_name="c", subcore_axis_name="s")

    @functools.partial(
        pl.kernel,
        out_type=jax.ShapeDtypeStruct((CTX, DIM), jnp.float32),
        mesh=mesh,
        scratch_types=[
            pltpu.VMEM((NCHUNK, CHUNK), jnp.int32),
            pltpu.VMEM((CHUNK, DIM), jnp.float32),
            pltpu.VMEM((DIM,), jnp.float32),
            pltpu.SemaphoreType.DMA,
        ],
        compiler_params=pltpu.CompilerParams(use_tc_tiling_on_sc=False),
    )
    def k(idx_hbm, table_hbm, out_hbm, idx_v, rows_v, acc_v, sem):
        wid = lax.axis_index("s") * 2 + lax.axis_index("c")

        @pl.when(wid < CTX)
        def _():
            pltpu.sync_copy(idx_hbm.at[wid], idx_v)

            def chunk_body(kk, carry):
                a0, a1 = carry
                pltpu.async_copy(table_hbm.at[idx_v.at[kk]], rows_v, sem).wait()

                def row_body(i, c2):
                    b0, b1 = c2
                    return (b0 + rows_v[i, pl.ds(0, 16)],
                            b1 + rows_v[i, pl.ds(16, 16)])

                return lax.fori_loop(0, CHUNK, row_body, (a0, a1), unroll=4)

            z = jnp.zeros((16,), jnp.float32)
            a0, a1 = lax.fori_loop(0, NCHUNK, chunk_body, (z, z))
            acc_v[pl.ds(0, 16)] = a0
            acc_v[pl.ds(16, 16)] = a1
            pltpu.sync_copy(acc_v, out_hbm.at[wid])

    return k(idx_t, emb_table)


def _tc_logits(flat, W, b2):
    grid = (pl.cdiv(VOCAB, VT),)

    def body(flat_ref, w_ref, b_ref, out_ref, lse_ref, m_sc, s_sc):
        i = pl.program_id(0)

        @pl.when(i == 0)
        def _():
            m_sc[0] = -jnp.inf
            s_sc[0] = jnp.float32(0.0)

        logits = lax.dot_general(
            flat_ref[...], w_ref[...], (((1,), (1,)), ((), ())),
            preferred_element_type=jnp.float32) + b_ref[...]
        gidx = i * VT + lax.broadcasted_iota(jnp.int32, (1, VT), 1)
        lm = jnp.where(gidx < VOCAB, logits, -jnp.inf)
        m0 = m_sc[0]
        m1 = jnp.maximum(m0, jnp.max(lm))
        s_sc[0] = s_sc[0] * jnp.exp(m0 - m1) + jnp.sum(jnp.exp(lm - m1))
        m_sc[0] = m1
        out_ref[...] = logits

        @pl.when(i == pl.num_programs(0) - 1)
        def _():
            lse_ref[...] = jnp.broadcast_to(m_sc[0] + jnp.log(s_sc[0]), (1, 1))

    return pl.pallas_call(
        body,
        grid=grid,
        in_specs=[
            pl.BlockSpec((1, CTXDIM), lambda i: (0, 0)),
            pl.BlockSpec((VT, CTXDIM), lambda i: (i, 0)),
            pl.BlockSpec((1, VT), lambda i: (0, i)),
        ],
        out_specs=[
            pl.BlockSpec((1, VT), lambda i: (0, i)),
            pl.BlockSpec((1, 1), lambda i: (0, 0)),
        ],
        out_shape=[
            jax.ShapeDtypeStruct((1, VOCAB), jnp.float32),
            jax.ShapeDtypeStruct((1, 1), jnp.float32),
        ],
        scratch_shapes=[
            pltpu.SMEM((1,), jnp.float32),
            pltpu.SMEM((1,), jnp.float32),
        ],
    )(flat, W, b2)


def _tc_subtract(logits, lse):
    def body(l_ref, lse_ref, o_ref):
        o_ref[...] = l_ref[...] - lse_ref[0, 0]

    return pl.pallas_call(
        body,
        grid=(pl.cdiv(VOCAB, VTC),),
        in_specs=[
            pl.BlockSpec((1, VTC), lambda i: (0, i)),
            pl.BlockSpec((1, 1), lambda i: (0, 0)),
        ],
        out_specs=pl.BlockSpec((1, VTC), lambda i: (0, i)),
        out_shape=jax.ShapeDtypeStruct((1, VOCAB), jnp.float32),
    )(logits, lse)


def kernel(inputs, emb_table, W, b):
    idx_t = inputs.T.reshape(CTX, NCHUNK, CHUNK)
    summed = _sc_gather_sum(idx_t, emb_table)
    flat = summed.reshape(1, CTXDIM)
    logits, lse = _tc_logits(flat, W, b.reshape(1, VOCAB))
    return _tc_subtract(logits, lse)
